# baseline (speedup 1.0000x reference)
import jax
import jax.numpy as jnp
from jax.experimental import pallas as pl
from jax.experimental.pallas import tpu as pltpu

POSITIVE = 1e-6
K_CONF = 100
K_IOU = 50
N = 20000
PAD_N = 20480
NR = 160
NL = 128
GW = 32
GH = 32
GPIX = GW * GH


def _nms_kernel(c0_ref, c1_ref, loc_ref, mask_ref, priors_ref, out_ref,
                gauss_ref, rows_ref):
    f32 = jnp.float32
    lin = (jax.lax.broadcasted_iota(jnp.int32, (NR, NL), 0) * NL
           + jax.lax.broadcasted_iota(jnp.int32, (NR, NL), 1))
    c0 = c0_ref[0]
    c1 = c1_ref[0]
    m = jnp.maximum(c0, c1)
    e0 = jnp.exp(c0 - m)
    e1 = jnp.exp(c1 - m)
    p = jnp.where(lin < N, e1 / (e0 + e1), -1.0)

    gauss_ref[:, :] = jnp.zeros((128, GPIX), f32)

    lane1024 = jax.lax.broadcasted_iota(jnp.int32, (1, GPIX), 1)
    gx_pos = (lane1024 % GW).astype(f32) * (1.0 / (GW - 1))
    gy_pos = (lane1024 // GW).astype(f32) * (1.0 / (GH - 1))
    lane4 = jax.lax.broadcasted_iota(jnp.int32, (1, 4), 1)
    lane128 = jax.lax.broadcasted_iota(jnp.int32, (1, 128), 1)
    row128 = jax.lax.broadcasted_iota(jnp.int32, (128, 1), 0)

    def pick(vec, j):
        return jnp.sum(jnp.where(lane4 == j, vec, 0.0))

    def body1(t, p):
        mval = jnp.max(p)
        idx = jnp.min(jnp.where(p == mval, lin, PAD_N))
        lv = loc_ref[0, pl.ds(idx, 1), :]
        pv = priors_ref[pl.ds(idx, 1), :]
        mv = mask_ref[0, pl.ds(idx, 1), :]
        ev = jnp.exp(lv * 0.2)
        l0 = pick(lv, 0)
        l1 = pick(lv, 1)
        p0 = pick(pv, 0)
        p1 = pick(pv, 1)
        p2 = pick(pv, 2)
        p3 = pick(pv, 3)
        w = p2 * pick(ev, 2)
        h = p3 * pick(ev, 3)
        cx = p0 + l0 * 0.1 * p2
        cy = p1 + l1 * 0.1 * p3
        dx = 2.0 * (w * 0.5) ** 2 + POSITIVE
        dy = 2.0 * (h * 0.5) ** 2 + POSITIVE
        g = jnp.exp(-((gx_pos - cx) ** 2 / dx + (gy_pos - cy) ** 2 / dy))
        gauss_ref[pl.ds(t, 1), :] = g
        row = jnp.concatenate(
            [lv, mv, jnp.full((1, 1), mval, f32), pv], axis=1)
        rows_ref[pl.ds(t, 1), :] = row
        return jnp.where(lin == idx, -jnp.inf, p)

    jax.lax.fori_loop(0, K_CONF, body1, p)

    G = gauss_ref[:, :]
    S = jnp.sum(G, axis=1, keepdims=True)

    def body2(j, im):
        gj = gauss_ref[pl.ds(j, 1), :]
        inter = jnp.sum(jnp.minimum(G, gj), axis=1, keepdims=True)
        sj = jnp.sum(gj)
        union = S + sj - inter
        iou = inter / (union + POSITIVE)
        colmax = jnp.max(jnp.where(row128 < j, iou, 0.0))
        return jnp.where(lane128 == j, colmax, im)

    im0 = jnp.where(lane128 < K_CONF, 0.0, jnp.inf)
    im = jax.lax.fori_loop(0, K_CONF, body2, im0)

    def body3(t, im):
        mval = jnp.min(im)
        kidx = jnp.min(jnp.where(im == mval, lane128, 128))
        out_ref[0, pl.ds(t, 1), :] = rows_ref[pl.ds(kidx, 1), :]
        return jnp.where(lane128 == kidx, jnp.inf, im)

    jax.lax.fori_loop(0, K_IOU, body3, im)


def kernel(original, conf, loc, mask, priors):
    del original
    B = conf.shape[0]
    cp = jnp.pad(conf, ((0, 0), (0, PAD_N - N), (0, 0)))
    cp = cp.reshape(B, NR, NL, 2)
    c0 = cp[..., 0]
    c1 = cp[..., 1]
    return pl.pallas_call(
        _nms_kernel,
        grid=(B,),
        in_specs=[
            pl.BlockSpec((1, NR, NL), lambda b: (b, 0, 0)),
            pl.BlockSpec((1, NR, NL), lambda b: (b, 0, 0)),
            pl.BlockSpec((1, N, 4), lambda b: (b, 0, 0)),
            pl.BlockSpec((1, N, 32), lambda b: (b, 0, 0)),
            pl.BlockSpec((N, 4), lambda b: (0, 0)),
        ],
        out_specs=pl.BlockSpec((1, K_IOU, 41), lambda b: (b, 0, 0)),
        out_shape=jax.ShapeDtypeStruct((B, K_IOU, 41), jnp.float32),
        scratch_shapes=[
            pltpu.VMEM((128, GPIX), jnp.float32),
            pltpu.VMEM((128, 41), jnp.float32),
        ],
    )(c0, c1, loc, mask, priors)


# speedup vs baseline: 2.7439x; 2.7439x over previous
import jax
import jax.numpy as jnp
from jax.experimental import pallas as pl
from jax.experimental.pallas import tpu as pltpu

POSITIVE = 1e-6
K_CONF = 100
K_IOU = 50
N = 20000
PAD_N = 20480
NR = 160
NL = 128
GW = 32
GH = 32
GPIX = GW * GH
B = 4
CHUNK = 8


def _nms_kernel(c0_ref, c1_ref, pk_ref, out_ref, gauss_ref, rows_ref):
    f32 = jnp.float32
    lin3 = (jax.lax.broadcasted_iota(jnp.int32, (1, NR, NL), 1) * NL
            + jax.lax.broadcasted_iota(jnp.int32, (1, NR, NL), 2))
    biota = jax.lax.broadcasted_iota(jnp.int32, (B, 1, 1), 0)
    lane3 = jax.lax.broadcasted_iota(jnp.int32, (1, 1, NL), 2)

    c0 = c0_ref[:, :, :]
    c1 = c1_ref[:, :, :]
    m = jnp.maximum(c0, c1)
    e0 = jnp.exp(c0 - m)
    e1 = jnp.exp(c1 - m)
    p = jnp.where(lin3 < N, e1 / (e0 + e1), -1.0)

    rows_ref[:, :, :] = jnp.zeros((B, 128, 48), f32)

    def body1(t, p):
        mvals = jnp.max(jnp.max(p, axis=2, keepdims=True), axis=1,
                        keepdims=True)
        cand = jnp.where(p == mvals, lin3, PAD_N)
        idxs = jnp.min(jnp.min(cand, axis=2, keepdims=True), axis=1,
                       keepdims=True)
        for b in range(B):
            idx_b = jnp.sum(jnp.where(biota == b, idxs, 0))
            mval_b = jnp.sum(jnp.where(biota == b, mvals, 0.0))
            pr = pk_ref[b, pl.ds(idx_b, 1), :]
            row = jnp.concatenate(
                [pr[:, 0:36], jnp.full((1, 1), mval_b, f32), pr[:, 36:40],
                 jnp.zeros((1, 7), f32)], axis=1)
            rows_ref[b, pl.ds(t, 1), :] = row
        return jnp.where(lin3 == idxs, -jnp.inf, p)

    jax.lax.fori_loop(0, K_CONF, body1, p)

    gx_pos = (jax.lax.broadcasted_iota(jnp.int32, (1, 1, GPIX), 2) % GW
              ).astype(f32) * (1.0 / (GW - 1))
    gy_pos = (jax.lax.broadcasted_iota(jnp.int32, (1, 1, GPIX), 2) // GW
              ).astype(f32) * (1.0 / (GH - 1))
    l0 = rows_ref[:, :, 0:1]
    l1 = rows_ref[:, :, 1:2]
    l2 = rows_ref[:, :, 2:3]
    l3 = rows_ref[:, :, 3:4]
    p0 = rows_ref[:, :, 37:38]
    p1 = rows_ref[:, :, 38:39]
    p2 = rows_ref[:, :, 39:40]
    p3 = rows_ref[:, :, 40:41]
    w = p2 * jnp.exp(l2 * 0.2)
    h = p3 * jnp.exp(l3 * 0.2)
    cx = p0 + l0 * 0.1 * p2
    cy = p1 + l1 * 0.1 * p3
    rdx = 1.0 / (2.0 * (w * 0.5) ** 2 + POSITIVE)
    rdy = 1.0 / (2.0 * (h * 0.5) ** 2 + POSITIVE)
    g = jnp.exp(-((gx_pos - cx) ** 2 * rdx + (gy_pos - cy) ** 2 * rdy))
    gauss_ref[:, :, :] = g
    S = jnp.sum(g, axis=2, keepdims=True)

    im = jnp.where(lane3 < K_CONF, jnp.zeros((B, 1, NL), f32), jnp.inf)
    for c in range(K_CONF // CHUNK + 1):
        j_lo = c * CHUNK
        n_cols = min(CHUNK, K_CONF - j_lo)
        if n_cols <= 0:
            break
        rc = min(128, (c + 1) * CHUNK)
        Gc = gauss_ref[:, 0:rc, :]
        Sc = S[:, 0:rc, :]
        rowc = jax.lax.broadcasted_iota(jnp.int32, (1, rc, 1), 1)

        def body2(jj, im, j_lo=j_lo, Gc=Gc, Sc=Sc, rowc=rowc):
            j = j_lo + jj
            gj = gauss_ref[:, pl.ds(j, 1), :]
            inter = jnp.sum(jnp.minimum(Gc, gj), axis=2, keepdims=True)
            sj = jnp.sum(gj, axis=2, keepdims=True)
            union = Sc + sj - inter
            iou = inter / (union + POSITIVE)
            masked = jnp.where(rowc < j, iou, 0.0)
            colmax = jnp.max(jnp.max(masked, axis=2, keepdims=True), axis=1,
                             keepdims=True)
            return jnp.where(lane3 == j, colmax, im)

        im = jax.lax.fori_loop(0, n_cols, body2, im)

    def body3(t, im):
        mvals = jnp.min(jnp.min(im, axis=2, keepdims=True), axis=1,
                        keepdims=True)
        cand = jnp.where(im == mvals, lane3, NL)
        kidx = jnp.min(jnp.min(cand, axis=2, keepdims=True), axis=1,
                       keepdims=True)
        for b in range(B):
            k_b = jnp.sum(jnp.where(biota == b, kidx, 0))
            out_ref[b, pl.ds(t, 1), :] = rows_ref[b, pl.ds(k_b, 1), 0:41]
        return jnp.where(lane3 == kidx, jnp.inf, im)

    jax.lax.fori_loop(0, K_IOU, body3, im)


def kernel(original, conf, loc, mask, priors):
    del original
    cp = jnp.pad(conf, ((0, 0), (0, PAD_N - N), (0, 0)))
    cp = cp.reshape(B, NR, NL, 2)
    c0 = cp[..., 0]
    c1 = cp[..., 1]
    priors_b = jnp.broadcast_to(priors[None], (B,) + priors.shape)
    packed = jnp.concatenate([loc, mask, priors_b], axis=2)
    return pl.pallas_call(
        _nms_kernel,
        in_specs=[
            pl.BlockSpec((B, NR, NL), lambda: (0, 0, 0)),
            pl.BlockSpec((B, NR, NL), lambda: (0, 0, 0)),
            pl.BlockSpec((B, N, 40), lambda: (0, 0, 0)),
        ],
        out_specs=pl.BlockSpec((B, K_IOU, 41), lambda: (0, 0, 0)),
        out_shape=jax.ShapeDtypeStruct((B, K_IOU, 41), jnp.float32),
        scratch_shapes=[
            pltpu.VMEM((B, 128, GPIX), jnp.float32),
            pltpu.VMEM((B, 128, 48), jnp.float32),
        ],
    )(c0, c1, packed)
